# Initial kernel scaffold; baseline (speedup 1.0000x reference)
import jax
import jax.numpy as jnp
from jax import lax
from jax.experimental import pallas as pl
from jax.experimental.pallas import tpu as pltpu
from jax.experimental.pallas import tpu_sc as plsc

N = 10000
E = 320000
D = 128

NPAD = 10240
COL = NPAD // 16
ROWS = 2528
EPAD = ROWS * 128
RD = ROWS // 16
RM = ROWS // 32


def _matvec_body(x_ref, w_ref, h_ref):
    h_ref[...] = jnp.sum(x_ref[...] * w_ref[...], axis=1, keepdims=True)


def _final_body(acc_ref, deg_ref, h_ref, b_ref, o_ref):
    acc = acc_ref[0] + acc_ref[1]
    dis = lax.rsqrt(deg_ref[...])
    g = h_ref[...] * dis
    z = dis * (acc + g) + b_ref[0:1, 0:1]
    o_ref[...] = jnp.maximum(z, 0.0) + jnp.log1p(jnp.exp(-jnp.abs(z)))


def _rsqrt16(d):
    i = plsc.bitcast(d, jnp.int32)
    i = jnp.full((16,), 0x5F3759DF, jnp.int32) - (i >> 1)
    y = plsc.bitcast(i, jnp.float32)
    for _ in range(3):
        y = y * (1.5 - 0.5 * d * y * y)
    return y


def _sc_body(h_hbm, src_hbm, dst_hbm, acc_hbm, deg_hbm,
             dstd_v, srcm_v, dstm_v, g_v, msg_v, col_v, hcol_v, ones_v,
             deg_sh, acc_sh, g_sh):
    cid = lax.axis_index("c")
    sid = lax.axis_index("s")
    base = sid * COL

    for i in range(COL // 16):
        col_v[pl.ds(i * 16, 16)] = jnp.zeros((16,), jnp.float32)
    pltpu.sync_copy(col_v, acc_sh.at[pl.ds(base, COL)])
    for i in range(COL // 16):
        col_v[pl.ds(i * 16, 16)] = jnp.ones((16,), jnp.float32)
    pltpu.sync_copy(col_v, deg_sh.at[pl.ds(base, COL)])
    for i in range(8):
        ones_v[pl.ds(i * 16, 16)] = jnp.ones((16,), jnp.float32)

    pltpu.sync_copy(dst_hbm.at[pl.ds(sid * RD, RD)], dstd_v)
    wid = cid * 16 + sid
    pltpu.sync_copy(src_hbm.at[pl.ds(wid * RM * 128, RM * 128)], srcm_v)
    pltpu.sync_copy(dst_hbm.at[pl.ds(wid * RM, RM)], dstm_v)
    plsc.subcore_barrier()

    @pl.loop(0, RD)
    def _deg(r):
        pltpu.sync_copy(ones_v, deg_sh.at[dstd_v.at[r]], add=True)

    plsc.subcore_barrier()

    pltpu.sync_copy(deg_sh.at[pl.ds(base, COL)], col_v)
    pltpu.sync_copy(h_hbm.at[pl.ds(base, COL)], hcol_v)
    for i in range(COL // 16):
        sl = pl.ds(i * 16, 16)
        hcol_v[sl] = hcol_v[sl] * _rsqrt16(col_v[sl])
    pltpu.sync_copy(hcol_v, g_sh.at[pl.ds(base, COL)])
    plsc.subcore_barrier()
    pltpu.sync_copy(g_sh, g_v)

    @pl.loop(0, RM)
    def _msg(r):
        for j in range(8):
            idx = srcm_v[pl.ds(r * 128 + j * 16, 16)]
            msg_v[r, pl.ds(j * 16, 16)] = plsc.load_gather(g_v, [idx])
        pltpu.sync_copy(msg_v.at[r], acc_sh.at[dstm_v.at[r]], add=True)

    plsc.subcore_barrier()

    @pl.when(sid == 0)
    def _():
        pltpu.sync_copy(acc_sh, acc_hbm.at[cid])

    @pl.when(jnp.logical_and(sid == 0, cid == 0))
    def _():
        pltpu.sync_copy(deg_sh, deg_hbm)


_sc_call = pl.kernel(
    _sc_body,
    out_type=(
        jax.ShapeDtypeStruct((2, NPAD), jnp.float32),
        jax.ShapeDtypeStruct((NPAD,), jnp.float32),
    ),
    mesh=plsc.VectorSubcoreMesh(core_axis_name="c", subcore_axis_name="s"),
    scratch_types=[
        pltpu.VMEM((RD, 128), jnp.int32),
        pltpu.VMEM((RM * 128,), jnp.int32),
        pltpu.VMEM((RM, 128), jnp.int32),
        pltpu.VMEM((NPAD,), jnp.float32),
        pltpu.VMEM((RM, 128), jnp.float32),
        pltpu.VMEM((COL,), jnp.float32),
        pltpu.VMEM((COL,), jnp.float32),
        pltpu.VMEM((128,), jnp.float32),
        pltpu.VMEM_SHARED((NPAD,), jnp.float32),
        pltpu.VMEM_SHARED((NPAD,), jnp.float32),
        pltpu.VMEM_SHARED((NPAD,), jnp.float32),
    ],
)


@jax.jit
def kernel(x, edge_index, W, b):
    h = pl.pallas_call(
        _matvec_body,
        out_shape=jax.ShapeDtypeStruct((N, 1), jnp.float32),
    )(x, W.reshape(1, D))

    h_pad = jnp.pad(h[:, 0], (0, NPAD - N))
    pad = jnp.full((EPAD - E,), N, jnp.int32)
    src_pad = jnp.concatenate([edge_index[0], pad])
    dst_pad = jnp.concatenate([edge_index[1], pad]).reshape(ROWS, 128)

    acc, deg = _sc_call(h_pad, src_pad, dst_pad)

    out2d = pl.pallas_call(
        _final_body,
        out_shape=jax.ShapeDtypeStruct((NPAD // 128, 128), jnp.float32),
    )(
        acc.reshape(2, NPAD // 128, 128),
        deg.reshape(NPAD // 128, 128),
        h_pad.reshape(NPAD // 128, 128),
        jnp.broadcast_to(b.reshape(1, 1), (8, 128)),
    )
    return out2d.reshape(NPAD)[:N].reshape(N, 1)


# speedup vs baseline: 89.7965x; 89.7965x over previous
import jax
import jax.numpy as jnp
from jax import lax
from jax.experimental import pallas as pl
from jax.experimental.pallas import tpu as pltpu
from jax.experimental.pallas import tpu_sc as plsc

N = 10000
E = 320000
D = 128

NPAD = 10240
COL = NPAD // 16
ROWS = 2560
EPAD = ROWS * 128
RD = ROWS // 16
RM = ROWS // 32


def _matvec_body(x_ref, w_ref, h_ref):
    h_ref[...] = jnp.sum(x_ref[...] * w_ref[...], axis=1, keepdims=True)


def _final_body(acc_ref, deg_ref, h_ref, b_ref, o_ref):
    acc = acc_ref[0] + acc_ref[1]
    dis = lax.rsqrt(deg_ref[...])
    g = h_ref[...] * dis
    z = dis * (acc + g) + b_ref[0:1, 0:1]
    o_ref[...] = jnp.maximum(z, 0.0) + jnp.log1p(jnp.exp(-jnp.abs(z)))


def _rsqrt16(d):
    i = plsc.bitcast(d, jnp.int32)
    i = jnp.full((16,), 0x5F3759DF, jnp.int32) - (i >> 1)
    y = plsc.bitcast(i, jnp.float32)
    for _ in range(3):
        y = y * (1.5 - 0.5 * d * y * y)
    return y


def _sc_body(h_hbm, src_hbm, dst_hbm, acc_hbm, deg_hbm,
             dstd_v, srcm_v, dstm_v, g_v, msg_v, col_v, hcol_v, ones_v,
             deg_sh, acc_sh, g_sh):
    cid = lax.axis_index("c")
    sid = lax.axis_index("s")
    base = sid * COL

    for i in range(COL // 16):
        col_v[pl.ds(i * 16, 16)] = jnp.zeros((16,), jnp.float32)
    pltpu.sync_copy(col_v, acc_sh.at[pl.ds(base, COL)])
    for i in range(COL // 16):
        col_v[pl.ds(i * 16, 16)] = jnp.ones((16,), jnp.float32)
    pltpu.sync_copy(col_v, deg_sh.at[pl.ds(base, COL)])
    for i in range(8):
        ones_v[pl.ds(i * 16, 16)] = jnp.ones((16,), jnp.float32)

    pltpu.sync_copy(dst_hbm.at[pl.ds(sid * RD, RD)], dstd_v)
    wid = cid * 16 + sid
    pltpu.sync_copy(src_hbm.at[pl.ds(wid * RM * 128, RM * 128)], srcm_v)
    pltpu.sync_copy(dst_hbm.at[pl.ds(wid * RM, RM)], dstm_v)
    plsc.subcore_barrier()

    @pl.loop(0, RD)
    def _deg(r):
        pltpu.sync_copy(ones_v, deg_sh.at[dstd_v.at[r]], add=True)

    plsc.subcore_barrier()

    pltpu.sync_copy(deg_sh.at[pl.ds(base, COL)], col_v)
    pltpu.sync_copy(h_hbm.at[pl.ds(base, COL)], hcol_v)
    for i in range(COL // 16):
        sl = pl.ds(i * 16, 16)
        hcol_v[sl] = hcol_v[sl] * _rsqrt16(col_v[sl])
    pltpu.sync_copy(hcol_v, g_sh.at[pl.ds(base, COL)])
    plsc.subcore_barrier()
    pltpu.sync_copy(g_sh, g_v)

    @pl.loop(0, RM)
    def _msg(r):
        for j in range(8):
            idx = srcm_v[pl.ds(r * 128 + j * 16, 16)]
            msg_v[r, pl.ds(j * 16, 16)] = plsc.load_gather(g_v, [idx])
        pltpu.sync_copy(msg_v.at[r], acc_sh.at[dstm_v.at[r]], add=True)

    plsc.subcore_barrier()

    @pl.when(sid == 0)
    def _():
        pltpu.sync_copy(acc_sh, acc_hbm.at[cid])

    @pl.when(jnp.logical_and(sid == 0, cid == 0))
    def _():
        pltpu.sync_copy(deg_sh, deg_hbm)


_sc_call = pl.kernel(
    _sc_body,
    out_type=(
        jax.ShapeDtypeStruct((2, NPAD), jnp.float32),
        jax.ShapeDtypeStruct((NPAD,), jnp.float32),
    ),
    mesh=plsc.VectorSubcoreMesh(core_axis_name="c", subcore_axis_name="s"),
    compiler_params=pltpu.CompilerParams(needs_layout_passes=False),
    scratch_types=[
        pltpu.VMEM((RD, 128), jnp.int32),
        pltpu.VMEM((RM * 128,), jnp.int32),
        pltpu.VMEM((RM, 128), jnp.int32),
        pltpu.VMEM((NPAD,), jnp.float32),
        pltpu.VMEM((RM, 128), jnp.float32),
        pltpu.VMEM((COL,), jnp.float32),
        pltpu.VMEM((COL,), jnp.float32),
        pltpu.VMEM((128,), jnp.float32),
        pltpu.VMEM_SHARED((NPAD,), jnp.float32),
        pltpu.VMEM_SHARED((NPAD,), jnp.float32),
        pltpu.VMEM_SHARED((NPAD,), jnp.float32),
    ],
)


@jax.jit
def kernel(x, edge_index, W, b):
    h = pl.pallas_call(
        _matvec_body,
        out_shape=jax.ShapeDtypeStruct((N, 1), jnp.float32),
    )(x, W.reshape(1, D))

    h_pad = jnp.pad(h[:, 0], (0, NPAD - N))
    pad = jnp.full((EPAD - E,), N, jnp.int32)
    src_pad = jnp.concatenate([edge_index[0], pad])
    dst_pad = jnp.concatenate([edge_index[1], pad]).reshape(ROWS, 128)

    acc, deg = _sc_call(h_pad, src_pad, dst_pad)

    out2d = pl.pallas_call(
        _final_body,
        out_shape=jax.ShapeDtypeStruct((NPAD // 128, 128), jnp.float32),
    )(
        acc.reshape(2, NPAD // 128, 128),
        deg.reshape(NPAD // 128, 128),
        h_pad.reshape(NPAD // 128, 128),
        jnp.broadcast_to(b.reshape(1, 1), (8, 128)),
    )
    return out2d.reshape(NPAD)[:N].reshape(N, 1)
